# Initial kernel scaffold; baseline (speedup 1.0000x reference)
import jax
import jax.numpy as jnp
from jax.experimental import pallas as pl


def kernel(transpose_8, getitem_3, view_4):
    raise NotImplementedError("write your pallas kernel here")


# speedup vs baseline: 1.2986x; 1.2986x over previous
import jax
import jax.numpy as jnp
from jax.experimental import pallas as pl
from jax.experimental.pallas import tpu as pltpu


def _add_window_kernel(t8_ref, g3_ref, out_ref):
    out_ref[...] = t8_ref[...]

    @pl.when(pl.program_id(1) == 0)
    def _():
        for h in range(12):
            out_ref[0, 1:256, h, 1:256] += g3_ref[0, h, :, :]


def kernel(transpose_8, getitem_3, view_4):
    del view_4
    g3 = getitem_3.reshape(4, 12, 255, 255)
    out = pl.pallas_call(
        _add_window_kernel,
        grid=(4, 4),
        in_specs=[
            pl.BlockSpec((1, 256, 12, 513), lambda b, j: (b, j, 0, 0)),
            pl.BlockSpec((1, 12, 255, 255), lambda b, j: (b, 0, 0, 0)),
        ],
        out_specs=pl.BlockSpec((1, 256, 12, 513), lambda b, j: (b, j, 0, 0)),
        out_shape=jax.ShapeDtypeStruct((4, 1024, 12, 513), transpose_8.dtype),
        compiler_params=pltpu.CompilerParams(
            dimension_semantics=("parallel", "arbitrary")
        ),
    )(transpose_8, g3)
    return (out,)
